# baseline (speedup 1.0000x reference)
import functools

import jax
import jax.numpy as jnp
from jax import lax
from jax.experimental import pallas as pl
from jax.experimental.pallas import tpu as pltpu
from jax.experimental.pallas import tpu_sc as plsc

NU = 50000
NR = 10000
D = 128
ER = 250000
EN = 100000

NUP = 50176
NRP = 10240
NR4 = 4 * NRP
ERP = 253952
ENP = 102400
NB_R = ERP // (32 * 128)
NB_N = ENP // (32 * 128)
CH_R = 31
CH_N = 25

_MESH = dict(core_axis_name="c", subcore_axis_name="s")


def _fill(ref, rows, width, value):
    vec = jnp.full((16,), value, jnp.float32)

    @pl.loop(0, rows)
    def _(i):
        for j in range(width // 16):
            ref[i, 16 * j:16 * (j + 1)] = vec


def _zero_shared(zsrc, acc, nchunks, s, chunk):
    @pl.loop(0, (nchunks + 15) // 16)
    def _(kk):
        ch = kk * 16 + s

        @pl.when(ch < nchunks)
        def _():
            pltpu.sync_copy(zsrc, acc.at[pl.ds(ch * chunk, chunk)])


def _sc_degrees(su, sd, ns, nd, z16):
    out_type = (
        jax.ShapeDtypeStruct((2, NUP, 16), jnp.float32),
        jax.ShapeDtypeStruct((2, NRP, 16), jnp.float32),
        jax.ShapeDtypeStruct((2, NRP, 16), jnp.float32),
        jax.ShapeDtypeStruct((2, NRP, 16), jnp.float32),
    )
    scratch = [
        pltpu.VMEM_SHARED((NUP, 16), jnp.float32),
        pltpu.VMEM_SHARED((NRP, 16), jnp.float32),
        pltpu.VMEM_SHARED((NRP, 16), jnp.float32),
        pltpu.VMEM_SHARED((NRP, 16), jnp.float32),
        pltpu.VMEM((128, 16), jnp.float32),
        pltpu.VMEM((NB_R, 128), jnp.int32),
        pltpu.SemaphoreType.DMA,
    ]

    @functools.partial(
        pl.kernel, out_type=out_type,
        mesh=plsc.VectorSubcoreMesh(**_MESH), scratch_types=scratch,
        compiler_params=pltpu.CompilerParams(use_tc_tiling_on_sc=False))
    def k(su_r, sd_r, ns_r, nd_r, z16_r, du_p, dr_p, dns_p, dnd_p,
          hu, hr1, hr2, hr3, ones_v, idx, sem):
        c = lax.axis_index("c")
        s = lax.axis_index("s")
        w = c * 16 + s
        _fill(ones_v, 128, 16, 1.0)
        _zero_shared(z16_r, hu, NUP // 512, s, 512)
        _zero_shared(z16_r, hr1, NRP // 512, s, 512)
        _zero_shared(z16_r, hr2, NRP // 512, s, 512)
        _zero_shared(z16_r, hr3, NRP // 512, s, 512)

        plsc.subcore_barrier()

        for arr, hist, nbw in ((su_r, hu, NB_R), (sd_r, hr1, NB_R),
                               (ns_r, hr2, NB_N), (nd_r, hr3, NB_N)):
            pltpu.sync_copy(arr.at[pl.ds(w * nbw, nbw)], idx.at[pl.ds(0, nbw)])

            @pl.loop(0, nbw, step=8)
            def _(b0, hist=hist, nbw=nbw):
                for j in range(8):
                    @pl.when(b0 + j < nbw)
                    def _(j=j):
                        pltpu.async_copy(ones_v, hist.at[idx.at[b0 + j]],
                                         sem, add=True)
                for j in range(8):
                    @pl.when(b0 + j < nbw)
                    def _(j=j):
                        pltpu.make_async_copy(
                            ones_v, hist.at[idx.at[b0 + j]], sem).wait()

        plsc.subcore_barrier()
        ru = NUP // 16
        rr = NRP // 16
        pltpu.sync_copy(hu.at[pl.ds(s * ru, ru)], du_p.at[c, pl.ds(s * ru, ru)])
        pltpu.sync_copy(hr1.at[pl.ds(s * rr, rr)], dr_p.at[c, pl.ds(s * rr, rr)])
        pltpu.sync_copy(hr2.at[pl.ds(s * rr, rr)], dns_p.at[c, pl.ds(s * rr, rr)])
        pltpu.sync_copy(hr3.at[pl.ds(s * rr, rr)], dnd_p.at[c, pl.ds(s * rr, rr)])

    return k(su, sd, ns, nd, z16)


def _agg_pipeline(tbl, idxg, idxs, rows, acc, gsems, ssems, nbw):
    pltpu.async_copy(tbl.at[idxg.at[0]], rows.at[0], gsems[0])

    @pl.loop(0, (nbw + 1) // 2, unroll=2)
    def _(t):
        for hb in (0, 1):
            b = 2 * t + hb

            @pl.when(b < nbw)
            def _(b=b, hb=hb):
                @pl.when(b + 1 < nbw)
                def _():
                    pltpu.async_copy(tbl.at[idxg.at[b + 1]],
                                     rows.at[1 - hb], gsems[1 - hb])
                pltpu.make_async_copy(tbl.at[idxg.at[b]],
                                      rows.at[hb], gsems[hb]).wait()
                pltpu.sync_copy(rows.at[hb], acc.at[idxs.at[b]], add=True)


def _sc_agg_layer(su, sd, ns, nd, ut, nt, rt):
    out_type = (
        jax.ShapeDtypeStruct((NR4, 32), jnp.float32),
        jax.ShapeDtypeStruct((NR4, 32), jnp.float32),
        jax.ShapeDtypeStruct((NR4, 32), jnp.float32),
        jax.ShapeDtypeStruct((NUP, 128), jnp.float32),
    )
    scratch = [
        pltpu.VMEM_SHARED((NUP, 32), jnp.float32),
        pltpu.VMEM((2, 128, 32), jnp.float32),
        pltpu.VMEM((CH_R, 128), jnp.int32),
        pltpu.VMEM((CH_R, 128), jnp.int32),
        pltpu.VMEM((128, 32), jnp.float32),
        pltpu.SemaphoreType.DMA,
        pltpu.SemaphoreType.DMA,
        pltpu.SemaphoreType.DMA,
        pltpu.SemaphoreType.DMA,
        pltpu.SemaphoreType.DMA,
        pltpu.SemaphoreType.DMA,
    ]

    @functools.partial(
        pl.kernel, out_type=out_type,
        mesh=plsc.VectorSubcoreMesh(**_MESH), scratch_types=scratch,
        compiler_params=pltpu.CompilerParams(use_tc_tiling_on_sc=False))
    def k(su_r, sd_r, ns_r, nd_r, ut_r, nt_r, rt_r,
          grev_o, gnear_a, gnear_b, gu_o,
          acc, rows, ia, ib, zb, g0, g1, g2, s0, s1, s2):
        c = lax.axis_index("c")
        s = lax.axis_index("s")
        gsems = (g0, g1, g2)
        ssems = (s0, s1, s2)
        _fill(zb, 128, 32, 0.0)

        def xform(ref, chw, scale, off):
            @pl.loop(0, chw)
            def _(r):
                for j in range(8):
                    sl = pl.ds(16 * j, 16)
                    if scale:
                        ref[r, sl] = ref[r, sl] * 4 + off
                    else:
                        ref[r, sl] = ref[r, sl] + 1

        def rest_agg(src_a, dst_a, tbl, out_ref, nch, chw, base0):
            _zero_shared(zb, acc, NR4 // 128, s, 128)
            plsc.subcore_barrier()
            for chunk in range(nch):
                base = base0 + s * (nch * chw) + chunk * chw
                pltpu.sync_copy(src_a.at[pl.ds(base, chw)],
                                ia.at[pl.ds(0, chw)])
                pltpu.sync_copy(dst_a.at[pl.ds(base, chw)],
                                ib.at[pl.ds(0, chw)])
                for p in range(4):
                    xform(ia, chw, p == 0, 0)
                    xform(ib, chw, p == 0, 0)
                    _agg_pipeline(tbl, ia, ib, rows, acc,
                                  gsems, ssems, chw)
            plsc.subcore_barrier()
            rr = NR4 // 16
            pltpu.sync_copy(acc.at[pl.ds(s * rr, rr)],
                            out_ref.at[pl.ds(s * rr, rr)])
            plsc.subcore_barrier()

        @pl.when(c == 0)
        def _():
            rest_agg(su_r, sd_r, ut_r, grev_o, 4, CH_R, 0)
            rest_agg(ns_r, nd_r, nt_r, gnear_a, 1, CH_N, 0)

        @pl.when(c == 1)
        def _():
            for p in range(4):
                _zero_shared(zb, acc, NUP // 128, s, 128)
                plsc.subcore_barrier()
                for chunk in range(4):
                    base = s * (4 * CH_R) + chunk * CH_R
                    pltpu.sync_copy(sd_r.at[pl.ds(base, CH_R)], ia)
                    pltpu.sync_copy(su_r.at[pl.ds(base, CH_R)], ib)
                    xform(ia, CH_R, True, p)
                    _agg_pipeline(rt_r, ia, ib, rows, acc,
                                  gsems, ssems, CH_R)
                plsc.subcore_barrier()
                ru = NUP // 16
                pltpu.sync_copy(acc.at[pl.ds(s * ru, ru)],
                                gu_o.at[pl.ds(s * ru, ru), pl.ds(32 * p, 32)])
                plsc.subcore_barrier()
            rest_agg(ns_r, nd_r, nt_r, gnear_b, 1, CH_N, 16 * CH_N)

    return k(su, sd, ns, nd, ut, nt, rt)


def _au_of(d_ref):
    d = d_ref[0] + d_ref[1]
    return lax.rsqrt(jnp.maximum(d[:, 0], 1.0))


def _an_of(d_ref):
    d = d_ref[0] + d_ref[1]
    return lax.rsqrt(d[:, 0] + 1.0)


_W_SPEC = pl.BlockSpec((128, 128), lambda i: (0, 0))
_D_SPEC = pl.BlockSpec((2, 512, 16), lambda i: (0, i, 0))
_ROW_SPEC = pl.BlockSpec((512, 128), lambda i: (i, 0))


def _tc_user_in(x, w, du_p):
    def body(x_ref, w_ref, d_ref, o_ref):
        au = _au_of(d_ref)
        h = jnp.dot(x_ref[...], w_ref[...], preferred_element_type=jnp.float32)
        o_ref[...] = h * au[:, None]

    return pl.pallas_call(
        body, grid=(NUP // 512,),
        in_specs=[_ROW_SPEC, _W_SPEC, _D_SPEC],
        out_specs=_ROW_SPEC,
        out_shape=jax.ShapeDtypeStruct((NUP, 128), jnp.float32),
    )(x, w, du_p)


def _tc_rest_in(x, w, dr_p, dns_p):
    def body(x_ref, w_ref, dr_ref, dns_ref, hr_ref, tn_ref, tr_ref):
        ar = _au_of(dr_ref)
        ans = _an_of(dns_ref)
        h = jnp.dot(x_ref[...], w_ref[...], preferred_element_type=jnp.float32)
        hr_ref[...] = h
        tn_ref[...] = h * ans[:, None]
        tr_ref[...] = h * ar[:, None]

    return pl.pallas_call(
        body, grid=(NRP // 512,),
        in_specs=[_ROW_SPEC, _W_SPEC, _D_SPEC, _D_SPEC],
        out_specs=[_ROW_SPEC] * 3,
        out_shape=[jax.ShapeDtypeStruct((NRP, 128), jnp.float32)] * 3,
    )(x, w, dr_p, dns_p)


def _tc_user_mid(gu, du_p, w_rev):
    def body(g_ref, d_ref, w_ref, o_ref):
        au = _au_of(d_ref)
        h = jnp.dot(g_ref[...], w_ref[...],
                    preferred_element_type=jnp.float32)
        hu = jnp.maximum(h * au[:, None], 0.0)
        o_ref[...] = hu * au[:, None]

    return pl.pallas_call(
        body, grid=(NUP // 512,),
        in_specs=[_ROW_SPEC, _D_SPEC, _W_SPEC],
        out_specs=_ROW_SPEC,
        out_shape=jax.ShapeDtypeStruct((NUP, 128), jnp.float32),
    )(gu, du_p, w_rev)


def _tc_user_out(gu, du_p, w_rev, w_out):
    def body(g_ref, d_ref, w_ref, wo_ref, o_ref):
        au = _au_of(d_ref)
        h = jnp.dot(g_ref[...], w_ref[...],
                    preferred_element_type=jnp.float32)
        hu = jnp.maximum(h * au[:, None], 0.0)
        o_ref[...] = jnp.dot(hu, wo_ref[...],
                             preferred_element_type=jnp.float32)

    return pl.pallas_call(
        body, grid=(NU // 400,),
        in_specs=[pl.BlockSpec((400, 128), lambda i: (i, 0)),
                  pl.BlockSpec((2, 400, 16), lambda i: (0, i, 0)),
                  _W_SPEC, _W_SPEC],
        out_specs=pl.BlockSpec((400, 128), lambda i: (i, 0)),
        out_shape=jax.ShapeDtypeStruct((NU, 128), jnp.float32),
    )(gu, du_p, w_rev, w_out)


def _rest_core(gr_ref, gna_ref, gnb_ref, hp_ref, dr_ref, dns_ref, dnd_ref,
               wr_ref, wn_ref):
    ar = _au_of(dr_ref)
    ans = _an_of(dns_ref)
    andd = _an_of(dnd_ref)
    hp = hp_ref[...]
    m1 = jnp.dot(gr_ref[...] * ar[:, None], wr_ref[...],
                 preferred_element_type=jnp.float32)
    gn = gna_ref[...] + gnb_ref[...]
    near_in = gn * andd[:, None] + hp * (ans * andd)[:, None]
    m2 = jnp.dot(near_in, wn_ref[...], preferred_element_type=jnp.float32)
    return jnp.maximum(m1 + m2, 0.0), ar, ans


_REST_IN_SPECS = [_ROW_SPEC, _ROW_SPEC, _ROW_SPEC, _ROW_SPEC,
                  _D_SPEC, _D_SPEC, _D_SPEC, _W_SPEC, _W_SPEC]


def _tc_rest_mid(grev, gna, gnb, hr_prev, dr_p, dns_p, dnd_p, w_rev, w_near):
    def body(gr_ref, gna_ref, gnb_ref, hp_ref, dr_ref, dns_ref, dnd_ref,
             wr_ref, wn_ref, hr_ref, tn_ref, tr_ref):
        hr, ar, ans = _rest_core(gr_ref, gna_ref, gnb_ref, hp_ref, dr_ref,
                                 dns_ref, dnd_ref, wr_ref, wn_ref)
        hr_ref[...] = hr
        tn_ref[...] = hr * ans[:, None]
        tr_ref[...] = hr * ar[:, None]

    return pl.pallas_call(
        body, grid=(NRP // 512,),
        in_specs=_REST_IN_SPECS,
        out_specs=[_ROW_SPEC] * 3,
        out_shape=[jax.ShapeDtypeStruct((NRP, 128), jnp.float32)] * 3,
    )(grev, gna, gnb, hr_prev, dr_p, dns_p, dnd_p, w_rev, w_near)


def _tc_rest_out(grev, gna, gnb, hr_prev, dr_p, dns_p, dnd_p,
                 w_rev, w_near, w_out):
    def body(gr_ref, gna_ref, gnb_ref, hp_ref, dr_ref, dns_ref, dnd_ref,
             wr_ref, wn_ref, wo_ref, o_ref):
        hr, _, _ = _rest_core(gr_ref, gna_ref, gnb_ref, hp_ref, dr_ref,
                              dns_ref, dnd_ref, wr_ref, wn_ref)
        o_ref[...] = jnp.dot(hr, wo_ref[...],
                             preferred_element_type=jnp.float32)

    rs = pl.BlockSpec((400, 128), lambda i: (i, 0))
    ds = pl.BlockSpec((2, 400, 16), lambda i: (0, i, 0))
    return pl.pallas_call(
        body, grid=(NR // 400,),
        in_specs=[rs, rs, rs, rs, ds, ds, ds, _W_SPEC, _W_SPEC, _W_SPEC],
        out_specs=rs,
        out_shape=jax.ShapeDtypeStruct((NR, 128), jnp.float32),
    )(grev, gna, gnb, hr_prev, dr_p, dns_p, dnd_p, w_rev, w_near, w_out)


def kernel(x_user, x_restaurant, W_in_user, W_in_rest, W1_reviews, W1_rev,
           W1_near, W2_reviews, W2_rev, W2_near, W_out_user, W_out_rest,
           edge_index_reviews, edge_index_rev_reviews, edge_index_near):
    i32 = jnp.int32
    su = edge_index_reviews[0].astype(i32)
    sd = edge_index_reviews[1].astype(i32)
    ns = edge_index_near[0].astype(i32)
    nd = edge_index_near[1].astype(i32)
    su = jnp.concatenate([su, jnp.full((ERP - ER,), NU, i32)]).reshape(-1, 128)
    sd = jnp.concatenate([sd, jnp.full((ERP - ER,), NR, i32)]).reshape(-1, 128)
    ns = jnp.concatenate([ns, jnp.full((ENP - EN,), NR, i32)]).reshape(-1, 128)
    nd = jnp.concatenate([nd, jnp.full((ENP - EN,), NR, i32)]).reshape(-1, 128)

    xu = jnp.pad(x_user, ((0, NUP - NU), (0, 0)))
    xr = jnp.pad(x_restaurant, ((0, NRP - NR), (0, 0)))

    z16 = jnp.zeros((512, 16), jnp.float32)

    du_p, dr_p, dns_p, dnd_p = _sc_degrees(su, sd, ns, nd, z16)

    c4 = lambda t: t.reshape(-1, 32)

    ut1 = _tc_user_in(xu, W_in_user, du_p)
    hr0, nt1, rt1 = _tc_rest_in(xr, W_in_rest, dr_p, dns_p)

    r128 = lambda t: t.reshape(NRP, 128)
    grev1, gna1, gnb1, gu1 = _sc_agg_layer(su, sd, ns, nd,
                                           c4(ut1), c4(nt1), c4(rt1))

    ut2 = _tc_user_mid(gu1, du_p, W1_rev)
    hr1, nt2, rt2 = _tc_rest_mid(
        r128(grev1), r128(gna1), r128(gnb1), hr0,
        dr_p, dns_p, dnd_p, W1_reviews, W1_near)

    grev2, gna2, gnb2, gu2 = _sc_agg_layer(su, sd, ns, nd,
                                           c4(ut2), c4(nt2), c4(rt2))

    out_u = _tc_user_out(gu2, du_p, W2_rev, W_out_user)
    out_r = _tc_rest_out(r128(grev2), r128(gna2), r128(gnb2),
                         hr1, dr_p, dns_p, dnd_p,
                         W2_reviews, W2_near, W_out_rest)

    return (out_u, out_r)


# speedup vs baseline: 1.0040x; 1.0040x over previous
import functools

import jax
import jax.numpy as jnp
from jax import lax
from jax.experimental import pallas as pl
from jax.experimental.pallas import tpu as pltpu
from jax.experimental.pallas import tpu_sc as plsc

NU = 50000
NR = 10000
D = 128
ER = 250000
EN = 100000

NUP = 50176
NRP = 10240
NR4 = 4 * NRP
ERP = 253952
ENP = 102400
NB_R = ERP // (32 * 128)
NB_N = ENP // (32 * 128)
CH_R = 31
CH_N = 25

_MESH = dict(core_axis_name="c", subcore_axis_name="s")


def _fill(ref, rows, width, value):
    vec = jnp.full((16,), value, jnp.float32)

    @pl.loop(0, rows)
    def _(i):
        for j in range(width // 16):
            ref[i, 16 * j:16 * (j + 1)] = vec


def _zero_shared(zsrc, acc, nchunks, s, chunk):
    @pl.loop(0, (nchunks + 15) // 16)
    def _(kk):
        ch = kk * 16 + s

        @pl.when(ch < nchunks)
        def _():
            pltpu.sync_copy(zsrc, acc.at[pl.ds(ch * chunk, chunk)])


def _sc_degrees(su, sd, ns, nd, z16):
    out_type = (
        jax.ShapeDtypeStruct((2, NUP, 16), jnp.float32),
        jax.ShapeDtypeStruct((2, NRP, 16), jnp.float32),
        jax.ShapeDtypeStruct((2, NRP, 16), jnp.float32),
        jax.ShapeDtypeStruct((2, NRP, 16), jnp.float32),
    )
    scratch = [
        pltpu.VMEM_SHARED((NUP, 16), jnp.float32),
        pltpu.VMEM_SHARED((NRP, 16), jnp.float32),
        pltpu.VMEM_SHARED((NRP, 16), jnp.float32),
        pltpu.VMEM_SHARED((NRP, 16), jnp.float32),
        pltpu.VMEM((128, 16), jnp.float32),
        pltpu.VMEM((NB_R, 128), jnp.int32),
        pltpu.SemaphoreType.DMA,
    ]

    @functools.partial(
        pl.kernel, out_type=out_type,
        mesh=plsc.VectorSubcoreMesh(**_MESH), scratch_types=scratch,
        compiler_params=pltpu.CompilerParams(use_tc_tiling_on_sc=False))
    def k(su_r, sd_r, ns_r, nd_r, z16_r, du_p, dr_p, dns_p, dnd_p,
          hu, hr1, hr2, hr3, ones_v, idx, sem):
        c = lax.axis_index("c")
        s = lax.axis_index("s")
        w = c * 16 + s
        _fill(ones_v, 128, 16, 1.0)
        _zero_shared(z16_r, hu, NUP // 512, s, 512)
        _zero_shared(z16_r, hr1, NRP // 512, s, 512)
        _zero_shared(z16_r, hr2, NRP // 512, s, 512)
        _zero_shared(z16_r, hr3, NRP // 512, s, 512)

        plsc.subcore_barrier()

        for arr, hist, nbw in ((su_r, hu, NB_R), (sd_r, hr1, NB_R),
                               (ns_r, hr2, NB_N), (nd_r, hr3, NB_N)):
            pltpu.sync_copy(arr.at[pl.ds(w * nbw, nbw)], idx.at[pl.ds(0, nbw)])

            @pl.loop(0, nbw, step=8)
            def _(b0, hist=hist, nbw=nbw):
                for j in range(8):
                    @pl.when(b0 + j < nbw)
                    def _(j=j):
                        pltpu.async_copy(ones_v, hist.at[idx.at[b0 + j]],
                                         sem, add=True)
                for j in range(8):
                    @pl.when(b0 + j < nbw)
                    def _(j=j):
                        pltpu.make_async_copy(
                            ones_v, hist.at[idx.at[b0 + j]], sem).wait()

        plsc.subcore_barrier()
        ru = NUP // 16
        rr = NRP // 16
        pltpu.sync_copy(hu.at[pl.ds(s * ru, ru)], du_p.at[c, pl.ds(s * ru, ru)])
        pltpu.sync_copy(hr1.at[pl.ds(s * rr, rr)], dr_p.at[c, pl.ds(s * rr, rr)])
        pltpu.sync_copy(hr2.at[pl.ds(s * rr, rr)], dns_p.at[c, pl.ds(s * rr, rr)])
        pltpu.sync_copy(hr3.at[pl.ds(s * rr, rr)], dnd_p.at[c, pl.ds(s * rr, rr)])

    return k(su, sd, ns, nd, z16)


def _agg_pipeline(tbl, idxg, idxs, rows, acc, gsems, ssems, nbw):
    pltpu.async_copy(tbl.at[idxg.at[0]], rows.at[0], gsems[0])

    @pl.loop(0, (nbw + 1) // 2)
    def _(t):
        for hb in (0, 1):
            b = 2 * t + hb

            @pl.when(b < nbw)
            def _(b=b, hb=hb):
                @pl.when(b + 1 < nbw)
                def _():
                    pltpu.async_copy(tbl.at[idxg.at[b + 1]],
                                     rows.at[1 - hb], gsems[1 - hb])
                pltpu.make_async_copy(tbl.at[idxg.at[b]],
                                      rows.at[hb], gsems[hb]).wait()
                pltpu.sync_copy(rows.at[hb], acc.at[idxs.at[b]], add=True)


def _sc_agg_layer(su, sd, ns, nd, ut, nt, rt):
    out_type = (
        jax.ShapeDtypeStruct((NR4, 32), jnp.float32),
        jax.ShapeDtypeStruct((NR4, 32), jnp.float32),
        jax.ShapeDtypeStruct((NR4, 32), jnp.float32),
        jax.ShapeDtypeStruct((NUP, 128), jnp.float32),
    )
    scratch = [
        pltpu.VMEM_SHARED((NUP, 32), jnp.float32),
        pltpu.VMEM((2, 128, 32), jnp.float32),
        pltpu.VMEM((CH_R, 128), jnp.int32),
        pltpu.VMEM((CH_R, 128), jnp.int32),
        pltpu.VMEM((128, 32), jnp.float32),
        pltpu.SemaphoreType.DMA,
        pltpu.SemaphoreType.DMA,
        pltpu.SemaphoreType.DMA,
        pltpu.SemaphoreType.DMA,
        pltpu.SemaphoreType.DMA,
        pltpu.SemaphoreType.DMA,
    ]

    @functools.partial(
        pl.kernel, out_type=out_type,
        mesh=plsc.VectorSubcoreMesh(**_MESH), scratch_types=scratch,
        compiler_params=pltpu.CompilerParams(use_tc_tiling_on_sc=False))
    def k(su_r, sd_r, ns_r, nd_r, ut_r, nt_r, rt_r,
          grev_o, gnear_a, gnear_b, gu_o,
          acc, rows, ia, ib, zb, g0, g1, g2, s0, s1, s2):
        c = lax.axis_index("c")
        s = lax.axis_index("s")
        gsems = (g0, g1, g2)
        ssems = (s0, s1, s2)
        _fill(zb, 128, 32, 0.0)

        def xform(ref, chw, scale, off):
            @pl.loop(0, chw)
            def _(r):
                for j in range(8):
                    sl = pl.ds(16 * j, 16)
                    if scale:
                        ref[r, sl] = ref[r, sl] * 4 + off
                    else:
                        ref[r, sl] = ref[r, sl] + 1

        def rest_agg(src_a, dst_a, tbl, out_ref, nch, chw, base0):
            _zero_shared(zb, acc, NR4 // 128, s, 128)
            plsc.subcore_barrier()
            for chunk in range(nch):
                base = base0 + s * (nch * chw) + chunk * chw
                pltpu.sync_copy(src_a.at[pl.ds(base, chw)],
                                ia.at[pl.ds(0, chw)])
                pltpu.sync_copy(dst_a.at[pl.ds(base, chw)],
                                ib.at[pl.ds(0, chw)])
                for p in range(4):
                    xform(ia, chw, p == 0, 0)
                    xform(ib, chw, p == 0, 0)
                    _agg_pipeline(tbl, ia, ib, rows, acc,
                                  gsems, ssems, chw)
            plsc.subcore_barrier()
            rr = NR4 // 16
            pltpu.sync_copy(acc.at[pl.ds(s * rr, rr)],
                            out_ref.at[pl.ds(s * rr, rr)])
            plsc.subcore_barrier()

        @pl.when(c == 0)
        def _():
            rest_agg(su_r, sd_r, ut_r, grev_o, 4, CH_R, 0)
            rest_agg(ns_r, nd_r, nt_r, gnear_a, 1, CH_N, 0)

        @pl.when(c == 1)
        def _():
            for p in range(4):
                _zero_shared(zb, acc, NUP // 128, s, 128)
                plsc.subcore_barrier()
                for chunk in range(4):
                    base = s * (4 * CH_R) + chunk * CH_R
                    pltpu.sync_copy(sd_r.at[pl.ds(base, CH_R)], ia)
                    pltpu.sync_copy(su_r.at[pl.ds(base, CH_R)], ib)
                    xform(ia, CH_R, True, p)
                    _agg_pipeline(rt_r, ia, ib, rows, acc,
                                  gsems, ssems, CH_R)
                plsc.subcore_barrier()
                ru = NUP // 16
                pltpu.sync_copy(acc.at[pl.ds(s * ru, ru)],
                                gu_o.at[pl.ds(s * ru, ru), pl.ds(32 * p, 32)])
                plsc.subcore_barrier()
            rest_agg(ns_r, nd_r, nt_r, gnear_b, 1, CH_N, 16 * CH_N)

    return k(su, sd, ns, nd, ut, nt, rt)


def _au_of(d_ref):
    d = d_ref[0] + d_ref[1]
    return lax.rsqrt(jnp.maximum(d[:, 0], 1.0))


def _an_of(d_ref):
    d = d_ref[0] + d_ref[1]
    return lax.rsqrt(d[:, 0] + 1.0)


_W_SPEC = pl.BlockSpec((128, 128), lambda i: (0, 0))
_D_SPEC = pl.BlockSpec((2, 512, 16), lambda i: (0, i, 0))
_ROW_SPEC = pl.BlockSpec((512, 128), lambda i: (i, 0))


def _tc_user_in(x, w, du_p):
    def body(x_ref, w_ref, d_ref, o_ref):
        au = _au_of(d_ref)
        h = jnp.dot(x_ref[...], w_ref[...], preferred_element_type=jnp.float32)
        o_ref[...] = h * au[:, None]

    return pl.pallas_call(
        body, grid=(NUP // 512,),
        in_specs=[_ROW_SPEC, _W_SPEC, _D_SPEC],
        out_specs=_ROW_SPEC,
        out_shape=jax.ShapeDtypeStruct((NUP, 128), jnp.float32),
    )(x, w, du_p)


def _tc_rest_in(x, w, dr_p, dns_p):
    def body(x_ref, w_ref, dr_ref, dns_ref, hr_ref, tn_ref, tr_ref):
        ar = _au_of(dr_ref)
        ans = _an_of(dns_ref)
        h = jnp.dot(x_ref[...], w_ref[...], preferred_element_type=jnp.float32)
        hr_ref[...] = h
        tn_ref[...] = h * ans[:, None]
        tr_ref[...] = h * ar[:, None]

    return pl.pallas_call(
        body, grid=(NRP // 512,),
        in_specs=[_ROW_SPEC, _W_SPEC, _D_SPEC, _D_SPEC],
        out_specs=[_ROW_SPEC] * 3,
        out_shape=[jax.ShapeDtypeStruct((NRP, 128), jnp.float32)] * 3,
    )(x, w, dr_p, dns_p)


def _tc_user_mid(gu, du_p, w_rev):
    def body(g_ref, d_ref, w_ref, o_ref):
        au = _au_of(d_ref)
        h = jnp.dot(g_ref[...], w_ref[...],
                    preferred_element_type=jnp.float32)
        hu = jnp.maximum(h * au[:, None], 0.0)
        o_ref[...] = hu * au[:, None]

    return pl.pallas_call(
        body, grid=(NUP // 512,),
        in_specs=[_ROW_SPEC, _D_SPEC, _W_SPEC],
        out_specs=_ROW_SPEC,
        out_shape=jax.ShapeDtypeStruct((NUP, 128), jnp.float32),
    )(gu, du_p, w_rev)


def _tc_user_out(gu, du_p, w_rev, w_out):
    def body(g_ref, d_ref, w_ref, wo_ref, o_ref):
        au = _au_of(d_ref)
        h = jnp.dot(g_ref[...], w_ref[...],
                    preferred_element_type=jnp.float32)
        hu = jnp.maximum(h * au[:, None], 0.0)
        o_ref[...] = jnp.dot(hu, wo_ref[...],
                             preferred_element_type=jnp.float32)

    return pl.pallas_call(
        body, grid=(NU // 400,),
        in_specs=[pl.BlockSpec((400, 128), lambda i: (i, 0)),
                  pl.BlockSpec((2, 400, 16), lambda i: (0, i, 0)),
                  _W_SPEC, _W_SPEC],
        out_specs=pl.BlockSpec((400, 128), lambda i: (i, 0)),
        out_shape=jax.ShapeDtypeStruct((NU, 128), jnp.float32),
    )(gu, du_p, w_rev, w_out)


def _rest_core(gr_ref, gna_ref, gnb_ref, hp_ref, dr_ref, dns_ref, dnd_ref,
               wr_ref, wn_ref):
    ar = _au_of(dr_ref)
    ans = _an_of(dns_ref)
    andd = _an_of(dnd_ref)
    hp = hp_ref[...]
    m1 = jnp.dot(gr_ref[...] * ar[:, None], wr_ref[...],
                 preferred_element_type=jnp.float32)
    gn = gna_ref[...] + gnb_ref[...]
    near_in = gn * andd[:, None] + hp * (ans * andd)[:, None]
    m2 = jnp.dot(near_in, wn_ref[...], preferred_element_type=jnp.float32)
    return jnp.maximum(m1 + m2, 0.0), ar, ans


_REST_IN_SPECS = [_ROW_SPEC, _ROW_SPEC, _ROW_SPEC, _ROW_SPEC,
                  _D_SPEC, _D_SPEC, _D_SPEC, _W_SPEC, _W_SPEC]


def _tc_rest_mid(grev, gna, gnb, hr_prev, dr_p, dns_p, dnd_p, w_rev, w_near):
    def body(gr_ref, gna_ref, gnb_ref, hp_ref, dr_ref, dns_ref, dnd_ref,
             wr_ref, wn_ref, hr_ref, tn_ref, tr_ref):
        hr, ar, ans = _rest_core(gr_ref, gna_ref, gnb_ref, hp_ref, dr_ref,
                                 dns_ref, dnd_ref, wr_ref, wn_ref)
        hr_ref[...] = hr
        tn_ref[...] = hr * ans[:, None]
        tr_ref[...] = hr * ar[:, None]

    return pl.pallas_call(
        body, grid=(NRP // 512,),
        in_specs=_REST_IN_SPECS,
        out_specs=[_ROW_SPEC] * 3,
        out_shape=[jax.ShapeDtypeStruct((NRP, 128), jnp.float32)] * 3,
    )(grev, gna, gnb, hr_prev, dr_p, dns_p, dnd_p, w_rev, w_near)


def _tc_rest_out(grev, gna, gnb, hr_prev, dr_p, dns_p, dnd_p,
                 w_rev, w_near, w_out):
    def body(gr_ref, gna_ref, gnb_ref, hp_ref, dr_ref, dns_ref, dnd_ref,
             wr_ref, wn_ref, wo_ref, o_ref):
        hr, _, _ = _rest_core(gr_ref, gna_ref, gnb_ref, hp_ref, dr_ref,
                              dns_ref, dnd_ref, wr_ref, wn_ref)
        o_ref[...] = jnp.dot(hr, wo_ref[...],
                             preferred_element_type=jnp.float32)

    rs = pl.BlockSpec((400, 128), lambda i: (i, 0))
    ds = pl.BlockSpec((2, 400, 16), lambda i: (0, i, 0))
    return pl.pallas_call(
        body, grid=(NR // 400,),
        in_specs=[rs, rs, rs, rs, ds, ds, ds, _W_SPEC, _W_SPEC, _W_SPEC],
        out_specs=rs,
        out_shape=jax.ShapeDtypeStruct((NR, 128), jnp.float32),
    )(grev, gna, gnb, hr_prev, dr_p, dns_p, dnd_p, w_rev, w_near, w_out)


def kernel(x_user, x_restaurant, W_in_user, W_in_rest, W1_reviews, W1_rev,
           W1_near, W2_reviews, W2_rev, W2_near, W_out_user, W_out_rest,
           edge_index_reviews, edge_index_rev_reviews, edge_index_near):
    i32 = jnp.int32
    su = edge_index_reviews[0].astype(i32)
    sd = edge_index_reviews[1].astype(i32)
    ns = edge_index_near[0].astype(i32)
    nd = edge_index_near[1].astype(i32)
    su = jnp.concatenate([su, jnp.full((ERP - ER,), NU, i32)]).reshape(-1, 128)
    sd = jnp.concatenate([sd, jnp.full((ERP - ER,), NR, i32)]).reshape(-1, 128)
    ns = jnp.concatenate([ns, jnp.full((ENP - EN,), NR, i32)]).reshape(-1, 128)
    nd = jnp.concatenate([nd, jnp.full((ENP - EN,), NR, i32)]).reshape(-1, 128)

    xu = jnp.pad(x_user, ((0, NUP - NU), (0, 0)))
    xr = jnp.pad(x_restaurant, ((0, NRP - NR), (0, 0)))

    z16 = jnp.zeros((512, 16), jnp.float32)

    du_p, dr_p, dns_p, dnd_p = _sc_degrees(su, sd, ns, nd, z16)

    c4 = lambda t: t.reshape(-1, 32)

    ut1 = _tc_user_in(xu, W_in_user, du_p)
    hr0, nt1, rt1 = _tc_rest_in(xr, W_in_rest, dr_p, dns_p)

    r128 = lambda t: t.reshape(NRP, 128)
    grev1, gna1, gnb1, gu1 = _sc_agg_layer(su, sd, ns, nd,
                                           c4(ut1), c4(nt1), c4(rt1))

    ut2 = _tc_user_mid(gu1, du_p, W1_rev)
    hr1, nt2, rt2 = _tc_rest_mid(
        r128(grev1), r128(gna1), r128(gnb1), hr0,
        dr_p, dns_p, dnd_p, W1_reviews, W1_near)

    grev2, gna2, gnb2, gu2 = _sc_agg_layer(su, sd, ns, nd,
                                           c4(ut2), c4(nt2), c4(rt2))

    out_u = _tc_user_out(gu2, du_p, W2_rev, W_out_user)
    out_r = _tc_rest_out(r128(grev2), r128(gna2), r128(gnb2),
                         hr1, dr_p, dns_p, dnd_p,
                         W2_reviews, W2_near, W_out_rest)

    return (out_u, out_r)


# speedup vs baseline: 1.0717x; 1.0675x over previous
import functools

import jax
import jax.numpy as jnp
from jax import lax
from jax.experimental import pallas as pl
from jax.experimental.pallas import tpu as pltpu
from jax.experimental.pallas import tpu_sc as plsc

NU = 50000
NR = 10000
D = 128
ER = 250000
EN = 100000

NUP = 50176
NRP = 10240
NR4 = 4 * NRP
ERP = 253952
ENP = 102400
NB_R = ERP // (32 * 128)
NB_N = ENP // (32 * 128)
CH_R = 31
CH_N = 25

_MESH = dict(core_axis_name="c", subcore_axis_name="s")


def _fill(ref, rows, width, value):
    vec = jnp.full((16,), value, jnp.float32)

    @pl.loop(0, rows)
    def _(i):
        for j in range(width // 16):
            ref[i, 16 * j:16 * (j + 1)] = vec


def _zero_shared(zsrc, acc, nchunks, s, chunk):
    @pl.loop(0, (nchunks + 15) // 16)
    def _(kk):
        ch = kk * 16 + s

        @pl.when(ch < nchunks)
        def _():
            pltpu.sync_copy(zsrc, acc.at[pl.ds(ch * chunk, chunk)])


def _sc_degrees(su, sd, ns, nd, z16):
    out_type = (
        jax.ShapeDtypeStruct((2, NUP, 16), jnp.float32),
        jax.ShapeDtypeStruct((2, NRP, 16), jnp.float32),
        jax.ShapeDtypeStruct((2, NRP, 16), jnp.float32),
        jax.ShapeDtypeStruct((2, NRP, 16), jnp.float32),
    )
    scratch = [
        pltpu.VMEM_SHARED((NUP, 16), jnp.float32),
        pltpu.VMEM_SHARED((NRP, 16), jnp.float32),
        pltpu.VMEM_SHARED((NRP, 16), jnp.float32),
        pltpu.VMEM_SHARED((NRP, 16), jnp.float32),
        pltpu.VMEM((128, 16), jnp.float32),
        pltpu.VMEM((NB_R, 128), jnp.int32),
        pltpu.SemaphoreType.DMA,
    ]

    @functools.partial(
        pl.kernel, out_type=out_type,
        mesh=plsc.VectorSubcoreMesh(**_MESH), scratch_types=scratch,
        compiler_params=pltpu.CompilerParams(use_tc_tiling_on_sc=False))
    def k(su_r, sd_r, ns_r, nd_r, z16_r, du_p, dr_p, dns_p, dnd_p,
          hu, hr1, hr2, hr3, ones_v, idx, sem):
        c = lax.axis_index("c")
        s = lax.axis_index("s")
        w = c * 16 + s
        _fill(ones_v, 128, 16, 1.0)
        _zero_shared(z16_r, hu, NUP // 512, s, 512)
        _zero_shared(z16_r, hr1, NRP // 512, s, 512)
        _zero_shared(z16_r, hr2, NRP // 512, s, 512)
        _zero_shared(z16_r, hr3, NRP // 512, s, 512)

        plsc.subcore_barrier()

        for arr, hist, nbw in ((su_r, hu, NB_R), (sd_r, hr1, NB_R),
                               (ns_r, hr2, NB_N), (nd_r, hr3, NB_N)):
            pltpu.sync_copy(arr.at[pl.ds(w * nbw, nbw)], idx.at[pl.ds(0, nbw)])

            @pl.loop(0, nbw, step=8)
            def _(b0, hist=hist, nbw=nbw):
                for j in range(8):
                    @pl.when(b0 + j < nbw)
                    def _(j=j):
                        pltpu.async_copy(ones_v, hist.at[idx.at[b0 + j]],
                                         sem, add=True)
                for j in range(8):
                    @pl.when(b0 + j < nbw)
                    def _(j=j):
                        pltpu.make_async_copy(
                            ones_v, hist.at[idx.at[b0 + j]], sem).wait()

        plsc.subcore_barrier()
        ru = NUP // 16
        rr = NRP // 16
        pltpu.sync_copy(hu.at[pl.ds(s * ru, ru)], du_p.at[c, pl.ds(s * ru, ru)])
        pltpu.sync_copy(hr1.at[pl.ds(s * rr, rr)], dr_p.at[c, pl.ds(s * rr, rr)])
        pltpu.sync_copy(hr2.at[pl.ds(s * rr, rr)], dns_p.at[c, pl.ds(s * rr, rr)])
        pltpu.sync_copy(hr3.at[pl.ds(s * rr, rr)], dnd_p.at[c, pl.ds(s * rr, rr)])

    return k(su, sd, ns, nd, z16)


def _agg_pipeline(tbl, idxg, idxs, rows, acc, gsems, ssems, nbw):
    del ssems
    pltpu.async_copy(tbl.at[idxg.at[0]], rows.at[0], gsems[0])
    pltpu.async_copy(tbl.at[idxg.at[1]], rows.at[1], gsems[1])

    @pl.loop(0, (nbw + 2) // 3)
    def _(t):
        for k in range(3):
            b = 3 * t + k
            k2 = (k + 2) % 3

            @pl.when(b < nbw)
            def _(b=b, k=k, k2=k2):
                @pl.when(b + 2 < nbw)
                def _():
                    pltpu.async_copy(tbl.at[idxg.at[b + 2]],
                                     rows.at[k2], gsems[k2])
                pltpu.make_async_copy(tbl.at[idxg.at[b]],
                                      rows.at[k], gsems[k]).wait()
                pltpu.sync_copy(rows.at[k], acc.at[idxs.at[b]], add=True)


def _sc_agg_layer(su, sd, ns, nd, ut, nt, rt):
    out_type = (
        jax.ShapeDtypeStruct((NR4, 32), jnp.float32),
        jax.ShapeDtypeStruct((NR4, 32), jnp.float32),
        jax.ShapeDtypeStruct((NR4, 32), jnp.float32),
        jax.ShapeDtypeStruct((NUP, 128), jnp.float32),
    )
    scratch = [
        pltpu.VMEM_SHARED((NUP, 32), jnp.float32),
        pltpu.VMEM((3, 128, 32), jnp.float32),
        pltpu.VMEM((CH_R, 128), jnp.int32),
        pltpu.VMEM((CH_R, 128), jnp.int32),
        pltpu.VMEM((128, 32), jnp.float32),
        pltpu.SemaphoreType.DMA,
        pltpu.SemaphoreType.DMA,
        pltpu.SemaphoreType.DMA,
        pltpu.SemaphoreType.DMA,
        pltpu.SemaphoreType.DMA,
        pltpu.SemaphoreType.DMA,
    ]

    @functools.partial(
        pl.kernel, out_type=out_type,
        mesh=plsc.VectorSubcoreMesh(**_MESH), scratch_types=scratch,
        compiler_params=pltpu.CompilerParams(use_tc_tiling_on_sc=False))
    def k(su_r, sd_r, ns_r, nd_r, ut_r, nt_r, rt_r,
          grev_o, gnear_a, gnear_b, gu_o,
          acc, rows, ia, ib, zb, g0, g1, g2, s0, s1, s2):
        c = lax.axis_index("c")
        s = lax.axis_index("s")
        gsems = (g0, g1, g2)
        ssems = (s0, s1, s2)
        _fill(zb, 128, 32, 0.0)

        def xform(ref, chw, scale, off):
            @pl.loop(0, chw)
            def _(r):
                for j in range(8):
                    sl = pl.ds(16 * j, 16)
                    if scale:
                        ref[r, sl] = ref[r, sl] * 4 + off
                    else:
                        ref[r, sl] = ref[r, sl] + 1

        def rest_agg(src_a, dst_a, tbl, out_ref, nch, chw, base0):
            _zero_shared(zb, acc, NR4 // 128, s, 128)
            plsc.subcore_barrier()
            for chunk in range(nch):
                base = base0 + s * (nch * chw) + chunk * chw
                pltpu.sync_copy(src_a.at[pl.ds(base, chw)],
                                ia.at[pl.ds(0, chw)])
                pltpu.sync_copy(dst_a.at[pl.ds(base, chw)],
                                ib.at[pl.ds(0, chw)])
                for p in range(4):
                    xform(ia, chw, p == 0, 0)
                    xform(ib, chw, p == 0, 0)
                    _agg_pipeline(tbl, ia, ib, rows, acc,
                                  gsems, ssems, chw)
            plsc.subcore_barrier()
            rr = NR4 // 16
            pltpu.sync_copy(acc.at[pl.ds(s * rr, rr)],
                            out_ref.at[pl.ds(s * rr, rr)])
            plsc.subcore_barrier()

        @pl.when(c == 0)
        def _():
            rest_agg(su_r, sd_r, ut_r, grev_o, 4, CH_R, 0)
            rest_agg(ns_r, nd_r, nt_r, gnear_a, 1, CH_N, 0)

        @pl.when(c == 1)
        def _():
            for p in range(4):
                _zero_shared(zb, acc, NUP // 128, s, 128)
                plsc.subcore_barrier()
                for chunk in range(4):
                    base = s * (4 * CH_R) + chunk * CH_R
                    pltpu.sync_copy(sd_r.at[pl.ds(base, CH_R)], ia)
                    pltpu.sync_copy(su_r.at[pl.ds(base, CH_R)], ib)
                    xform(ia, CH_R, True, p)
                    _agg_pipeline(rt_r, ia, ib, rows, acc,
                                  gsems, ssems, CH_R)
                plsc.subcore_barrier()
                ru = NUP // 16
                pltpu.sync_copy(acc.at[pl.ds(s * ru, ru)],
                                gu_o.at[pl.ds(s * ru, ru), pl.ds(32 * p, 32)])
                plsc.subcore_barrier()
            rest_agg(ns_r, nd_r, nt_r, gnear_b, 1, CH_N, 16 * CH_N)

    return k(su, sd, ns, nd, ut, nt, rt)


def _au_of(d_ref):
    d = d_ref[0] + d_ref[1]
    return lax.rsqrt(jnp.maximum(d[:, 0], 1.0))


def _an_of(d_ref):
    d = d_ref[0] + d_ref[1]
    return lax.rsqrt(d[:, 0] + 1.0)


_W_SPEC = pl.BlockSpec((128, 128), lambda i: (0, 0))
_D_SPEC = pl.BlockSpec((2, 512, 16), lambda i: (0, i, 0))
_ROW_SPEC = pl.BlockSpec((512, 128), lambda i: (i, 0))


def _tc_user_in(x, w, du_p):
    def body(x_ref, w_ref, d_ref, o_ref):
        au = _au_of(d_ref)
        h = jnp.dot(x_ref[...], w_ref[...], preferred_element_type=jnp.float32)
        o_ref[...] = h * au[:, None]

    return pl.pallas_call(
        body, grid=(NUP // 512,),
        in_specs=[_ROW_SPEC, _W_SPEC, _D_SPEC],
        out_specs=_ROW_SPEC,
        out_shape=jax.ShapeDtypeStruct((NUP, 128), jnp.float32),
    )(x, w, du_p)


def _tc_rest_in(x, w, dr_p, dns_p):
    def body(x_ref, w_ref, dr_ref, dns_ref, hr_ref, tn_ref, tr_ref):
        ar = _au_of(dr_ref)
        ans = _an_of(dns_ref)
        h = jnp.dot(x_ref[...], w_ref[...], preferred_element_type=jnp.float32)
        hr_ref[...] = h
        tn_ref[...] = h * ans[:, None]
        tr_ref[...] = h * ar[:, None]

    return pl.pallas_call(
        body, grid=(NRP // 512,),
        in_specs=[_ROW_SPEC, _W_SPEC, _D_SPEC, _D_SPEC],
        out_specs=[_ROW_SPEC] * 3,
        out_shape=[jax.ShapeDtypeStruct((NRP, 128), jnp.float32)] * 3,
    )(x, w, dr_p, dns_p)


def _tc_user_mid(gu, du_p, w_rev):
    def body(g_ref, d_ref, w_ref, o_ref):
        au = _au_of(d_ref)
        h = jnp.dot(g_ref[...], w_ref[...],
                    preferred_element_type=jnp.float32)
        hu = jnp.maximum(h * au[:, None], 0.0)
        o_ref[...] = hu * au[:, None]

    return pl.pallas_call(
        body, grid=(NUP // 512,),
        in_specs=[_ROW_SPEC, _D_SPEC, _W_SPEC],
        out_specs=_ROW_SPEC,
        out_shape=jax.ShapeDtypeStruct((NUP, 128), jnp.float32),
    )(gu, du_p, w_rev)


def _tc_user_out(gu, du_p, w_rev, w_out):
    def body(g_ref, d_ref, w_ref, wo_ref, o_ref):
        au = _au_of(d_ref)
        h = jnp.dot(g_ref[...], w_ref[...],
                    preferred_element_type=jnp.float32)
        hu = jnp.maximum(h * au[:, None], 0.0)
        o_ref[...] = jnp.dot(hu, wo_ref[...],
                             preferred_element_type=jnp.float32)

    return pl.pallas_call(
        body, grid=(NU // 400,),
        in_specs=[pl.BlockSpec((400, 128), lambda i: (i, 0)),
                  pl.BlockSpec((2, 400, 16), lambda i: (0, i, 0)),
                  _W_SPEC, _W_SPEC],
        out_specs=pl.BlockSpec((400, 128), lambda i: (i, 0)),
        out_shape=jax.ShapeDtypeStruct((NU, 128), jnp.float32),
    )(gu, du_p, w_rev, w_out)


def _rest_core(gr_ref, gna_ref, gnb_ref, hp_ref, dr_ref, dns_ref, dnd_ref,
               wr_ref, wn_ref):
    ar = _au_of(dr_ref)
    ans = _an_of(dns_ref)
    andd = _an_of(dnd_ref)
    hp = hp_ref[...]
    m1 = jnp.dot(gr_ref[...] * ar[:, None], wr_ref[...],
                 preferred_element_type=jnp.float32)
    gn = gna_ref[...] + gnb_ref[...]
    near_in = gn * andd[:, None] + hp * (ans * andd)[:, None]
    m2 = jnp.dot(near_in, wn_ref[...], preferred_element_type=jnp.float32)
    return jnp.maximum(m1 + m2, 0.0), ar, ans


_REST_IN_SPECS = [_ROW_SPEC, _ROW_SPEC, _ROW_SPEC, _ROW_SPEC,
                  _D_SPEC, _D_SPEC, _D_SPEC, _W_SPEC, _W_SPEC]


def _tc_rest_mid(grev, gna, gnb, hr_prev, dr_p, dns_p, dnd_p, w_rev, w_near):
    def body(gr_ref, gna_ref, gnb_ref, hp_ref, dr_ref, dns_ref, dnd_ref,
             wr_ref, wn_ref, hr_ref, tn_ref, tr_ref):
        hr, ar, ans = _rest_core(gr_ref, gna_ref, gnb_ref, hp_ref, dr_ref,
                                 dns_ref, dnd_ref, wr_ref, wn_ref)
        hr_ref[...] = hr
        tn_ref[...] = hr * ans[:, None]
        tr_ref[...] = hr * ar[:, None]

    return pl.pallas_call(
        body, grid=(NRP // 512,),
        in_specs=_REST_IN_SPECS,
        out_specs=[_ROW_SPEC] * 3,
        out_shape=[jax.ShapeDtypeStruct((NRP, 128), jnp.float32)] * 3,
    )(grev, gna, gnb, hr_prev, dr_p, dns_p, dnd_p, w_rev, w_near)


def _tc_rest_out(grev, gna, gnb, hr_prev, dr_p, dns_p, dnd_p,
                 w_rev, w_near, w_out):
    def body(gr_ref, gna_ref, gnb_ref, hp_ref, dr_ref, dns_ref, dnd_ref,
             wr_ref, wn_ref, wo_ref, o_ref):
        hr, _, _ = _rest_core(gr_ref, gna_ref, gnb_ref, hp_ref, dr_ref,
                              dns_ref, dnd_ref, wr_ref, wn_ref)
        o_ref[...] = jnp.dot(hr, wo_ref[...],
                             preferred_element_type=jnp.float32)

    rs = pl.BlockSpec((400, 128), lambda i: (i, 0))
    ds = pl.BlockSpec((2, 400, 16), lambda i: (0, i, 0))
    return pl.pallas_call(
        body, grid=(NR // 400,),
        in_specs=[rs, rs, rs, rs, ds, ds, ds, _W_SPEC, _W_SPEC, _W_SPEC],
        out_specs=rs,
        out_shape=jax.ShapeDtypeStruct((NR, 128), jnp.float32),
    )(grev, gna, gnb, hr_prev, dr_p, dns_p, dnd_p, w_rev, w_near, w_out)


def kernel(x_user, x_restaurant, W_in_user, W_in_rest, W1_reviews, W1_rev,
           W1_near, W2_reviews, W2_rev, W2_near, W_out_user, W_out_rest,
           edge_index_reviews, edge_index_rev_reviews, edge_index_near):
    i32 = jnp.int32
    su = edge_index_reviews[0].astype(i32)
    sd = edge_index_reviews[1].astype(i32)
    ns = edge_index_near[0].astype(i32)
    nd = edge_index_near[1].astype(i32)
    su = jnp.concatenate([su, jnp.full((ERP - ER,), NU, i32)]).reshape(-1, 128)
    sd = jnp.concatenate([sd, jnp.full((ERP - ER,), NR, i32)]).reshape(-1, 128)
    ns = jnp.concatenate([ns, jnp.full((ENP - EN,), NR, i32)]).reshape(-1, 128)
    nd = jnp.concatenate([nd, jnp.full((ENP - EN,), NR, i32)]).reshape(-1, 128)

    xu = jnp.pad(x_user, ((0, NUP - NU), (0, 0)))
    xr = jnp.pad(x_restaurant, ((0, NRP - NR), (0, 0)))

    z16 = jnp.zeros((512, 16), jnp.float32)

    du_p, dr_p, dns_p, dnd_p = _sc_degrees(su, sd, ns, nd, z16)

    c4 = lambda t: t.reshape(-1, 32)

    ut1 = _tc_user_in(xu, W_in_user, du_p)
    hr0, nt1, rt1 = _tc_rest_in(xr, W_in_rest, dr_p, dns_p)

    r128 = lambda t: t.reshape(NRP, 128)
    grev1, gna1, gnb1, gu1 = _sc_agg_layer(su, sd, ns, nd,
                                           c4(ut1), c4(nt1), c4(rt1))

    ut2 = _tc_user_mid(gu1, du_p, W1_rev)
    hr1, nt2, rt2 = _tc_rest_mid(
        r128(grev1), r128(gna1), r128(gnb1), hr0,
        dr_p, dns_p, dnd_p, W1_reviews, W1_near)

    grev2, gna2, gnb2, gu2 = _sc_agg_layer(su, sd, ns, nd,
                                           c4(ut2), c4(nt2), c4(rt2))

    out_u = _tc_user_out(gu2, du_p, W2_rev, W_out_user)
    out_r = _tc_rest_out(r128(grev2), r128(gna2), r128(gnb2),
                         hr1, dr_p, dns_p, dnd_p,
                         W2_reviews, W2_near, W_out_rest)

    return (out_u, out_r)


# speedup vs baseline: 1.0865x; 1.0138x over previous
import functools

import jax
import jax.numpy as jnp
from jax import lax
from jax.experimental import pallas as pl
from jax.experimental.pallas import tpu as pltpu
from jax.experimental.pallas import tpu_sc as plsc

NU = 50000
NR = 10000
D = 128
ER = 250000
EN = 100000

NUP = 50176
NRP = 10240
NR4 = 4 * NRP
ERP = 253952
ENP = 102400
NB_R = ERP // (32 * 128)
NB_N = ENP // (32 * 128)
CH_R = 31
CH_N = 25

_MESH = dict(core_axis_name="c", subcore_axis_name="s")


def _fill(ref, rows, width, value):
    vec = jnp.full((16,), value, jnp.float32)

    @pl.loop(0, rows)
    def _(i):
        for j in range(width // 16):
            ref[i, 16 * j:16 * (j + 1)] = vec


def _zero_shared(zsrc, acc, nchunks, s, chunk):
    @pl.loop(0, (nchunks + 15) // 16)
    def _(kk):
        ch = kk * 16 + s

        @pl.when(ch < nchunks)
        def _():
            pltpu.sync_copy(zsrc, acc.at[pl.ds(ch * chunk, chunk)])


def _sc_degrees(su, sd, ns, nd, z16):
    out_type = (
        jax.ShapeDtypeStruct((2, NUP, 16), jnp.float32),
        jax.ShapeDtypeStruct((2, NRP, 16), jnp.float32),
        jax.ShapeDtypeStruct((2, NRP, 16), jnp.float32),
        jax.ShapeDtypeStruct((2, NRP, 16), jnp.float32),
    )
    scratch = [
        pltpu.VMEM_SHARED((NUP, 16), jnp.float32),
        pltpu.VMEM_SHARED((NRP, 16), jnp.float32),
        pltpu.VMEM_SHARED((NRP, 16), jnp.float32),
        pltpu.VMEM_SHARED((NRP, 16), jnp.float32),
        pltpu.VMEM((128, 16), jnp.float32),
        pltpu.VMEM((NB_R, 128), jnp.int32),
        pltpu.SemaphoreType.DMA,
    ]

    @functools.partial(
        pl.kernel, out_type=out_type,
        mesh=plsc.VectorSubcoreMesh(**_MESH), scratch_types=scratch,
        compiler_params=pltpu.CompilerParams(use_tc_tiling_on_sc=False))
    def k(su_r, sd_r, ns_r, nd_r, z16_r, du_p, dr_p, dns_p, dnd_p,
          hu, hr1, hr2, hr3, ones_v, idx, sem):
        c = lax.axis_index("c")
        s = lax.axis_index("s")
        w = c * 16 + s
        _fill(ones_v, 128, 16, 1.0)
        _zero_shared(z16_r, hu, NUP // 512, s, 512)
        _zero_shared(z16_r, hr1, NRP // 512, s, 512)
        _zero_shared(z16_r, hr2, NRP // 512, s, 512)
        _zero_shared(z16_r, hr3, NRP // 512, s, 512)

        plsc.subcore_barrier()

        for arr, hist, nbw in ((su_r, hu, NB_R), (sd_r, hr1, NB_R),
                               (ns_r, hr2, NB_N), (nd_r, hr3, NB_N)):
            pltpu.sync_copy(arr.at[pl.ds(w * nbw, nbw)], idx.at[pl.ds(0, nbw)])

            @pl.loop(0, nbw, step=8)
            def _(b0, hist=hist, nbw=nbw):
                for j in range(8):
                    @pl.when(b0 + j < nbw)
                    def _(j=j):
                        pltpu.async_copy(ones_v, hist.at[idx.at[b0 + j]],
                                         sem, add=True)
                for j in range(8):
                    @pl.when(b0 + j < nbw)
                    def _(j=j):
                        pltpu.make_async_copy(
                            ones_v, hist.at[idx.at[b0 + j]], sem).wait()

        plsc.subcore_barrier()
        ru = NUP // 16
        rr = NRP // 16
        pltpu.sync_copy(hu.at[pl.ds(s * ru, ru)], du_p.at[c, pl.ds(s * ru, ru)])
        pltpu.sync_copy(hr1.at[pl.ds(s * rr, rr)], dr_p.at[c, pl.ds(s * rr, rr)])
        pltpu.sync_copy(hr2.at[pl.ds(s * rr, rr)], dns_p.at[c, pl.ds(s * rr, rr)])
        pltpu.sync_copy(hr3.at[pl.ds(s * rr, rr)], dnd_p.at[c, pl.ds(s * rr, rr)])

    return k(su, sd, ns, nd, z16)


def _agg_pipeline(tbl, idxg, idxs, rows, acc, gsems, ssems, nbw):
    pltpu.async_copy(tbl.at[idxg.at[0]], rows.at[0], gsems[0])
    pltpu.async_copy(tbl.at[idxg.at[1]], rows.at[1], gsems[1])
    pltpu.async_copy(tbl.at[idxg.at[2]], rows.at[2], gsems[2])
    sems4 = gsems + (ssems[0],)

    @pl.loop(0, (nbw + 3) // 4)
    def _(t):
        for k in range(4):
            b = 4 * t + k
            k2 = (k + 3) % 4

            @pl.when(b < nbw)
            def _(b=b, k=k, k2=k2):
                @pl.when(b + 3 < nbw)
                def _():
                    pltpu.async_copy(tbl.at[idxg.at[b + 3]],
                                     rows.at[k2], sems4[k2])
                pltpu.make_async_copy(tbl.at[idxg.at[b]],
                                      rows.at[k], sems4[k]).wait()
                pltpu.sync_copy(rows.at[k], acc.at[idxs.at[b]], add=True)


def _sc_agg_layer(su, sd, ns, nd, ut, nt, rt):
    out_type = (
        jax.ShapeDtypeStruct((NR4, 32), jnp.float32),
        jax.ShapeDtypeStruct((NR4, 32), jnp.float32),
        jax.ShapeDtypeStruct((NR4, 32), jnp.float32),
        jax.ShapeDtypeStruct((NUP, 128), jnp.float32),
    )
    scratch = [
        pltpu.VMEM_SHARED((NUP, 32), jnp.float32),
        pltpu.VMEM((4, 128, 32), jnp.float32),
        pltpu.VMEM((CH_R, 128), jnp.int32),
        pltpu.VMEM((CH_R, 128), jnp.int32),
        pltpu.VMEM((128, 32), jnp.float32),
        pltpu.SemaphoreType.DMA,
        pltpu.SemaphoreType.DMA,
        pltpu.SemaphoreType.DMA,
        pltpu.SemaphoreType.DMA,
        pltpu.SemaphoreType.DMA,
        pltpu.SemaphoreType.DMA,
    ]

    @functools.partial(
        pl.kernel, out_type=out_type,
        mesh=plsc.VectorSubcoreMesh(**_MESH), scratch_types=scratch,
        compiler_params=pltpu.CompilerParams(use_tc_tiling_on_sc=False))
    def k(su_r, sd_r, ns_r, nd_r, ut_r, nt_r, rt_r,
          grev_o, gnear_a, gnear_b, gu_o,
          acc, rows, ia, ib, zb, g0, g1, g2, s0, s1, s2):
        c = lax.axis_index("c")
        s = lax.axis_index("s")
        gsems = (g0, g1, g2)
        ssems = (s0, s1, s2)
        _fill(zb, 128, 32, 0.0)

        def xform(ref, chw, scale, off):
            @pl.loop(0, chw)
            def _(r):
                for j in range(8):
                    sl = pl.ds(16 * j, 16)
                    if scale:
                        ref[r, sl] = ref[r, sl] * 4 + off
                    else:
                        ref[r, sl] = ref[r, sl] + 1

        def rest_agg(src_a, dst_a, tbl, out_ref, nch, chw, base0):
            _zero_shared(zb, acc, NR4 // 128, s, 128)
            plsc.subcore_barrier()
            for chunk in range(nch):
                base = base0 + s * (nch * chw) + chunk * chw
                pltpu.sync_copy(src_a.at[pl.ds(base, chw)],
                                ia.at[pl.ds(0, chw)])
                pltpu.sync_copy(dst_a.at[pl.ds(base, chw)],
                                ib.at[pl.ds(0, chw)])
                for p in range(4):
                    xform(ia, chw, p == 0, 0)
                    xform(ib, chw, p == 0, 0)
                    _agg_pipeline(tbl, ia, ib, rows, acc,
                                  gsems, ssems, chw)
            plsc.subcore_barrier()
            rr = NR4 // 16
            pltpu.sync_copy(acc.at[pl.ds(s * rr, rr)],
                            out_ref.at[pl.ds(s * rr, rr)])
            plsc.subcore_barrier()

        @pl.when(c == 0)
        def _():
            rest_agg(su_r, sd_r, ut_r, grev_o, 4, CH_R, 0)
            rest_agg(ns_r, nd_r, nt_r, gnear_a, 1, CH_N, 0)

        @pl.when(c == 1)
        def _():
            for p in range(4):
                _zero_shared(zb, acc, NUP // 128, s, 128)
                plsc.subcore_barrier()
                for chunk in range(4):
                    base = s * (4 * CH_R) + chunk * CH_R
                    pltpu.sync_copy(sd_r.at[pl.ds(base, CH_R)], ia)
                    pltpu.sync_copy(su_r.at[pl.ds(base, CH_R)], ib)
                    xform(ia, CH_R, True, p)
                    _agg_pipeline(rt_r, ia, ib, rows, acc,
                                  gsems, ssems, CH_R)
                plsc.subcore_barrier()
                ru = NUP // 16
                pltpu.sync_copy(acc.at[pl.ds(s * ru, ru)],
                                gu_o.at[pl.ds(s * ru, ru), pl.ds(32 * p, 32)])
                plsc.subcore_barrier()
            rest_agg(ns_r, nd_r, nt_r, gnear_b, 1, CH_N, 16 * CH_N)

    return k(su, sd, ns, nd, ut, nt, rt)


def _au_of(d_ref):
    d = d_ref[0] + d_ref[1]
    return lax.rsqrt(jnp.maximum(d[:, 0], 1.0))


def _an_of(d_ref):
    d = d_ref[0] + d_ref[1]
    return lax.rsqrt(d[:, 0] + 1.0)


_W_SPEC = pl.BlockSpec((128, 128), lambda i: (0, 0))
_D_SPEC = pl.BlockSpec((2, 512, 16), lambda i: (0, i, 0))
_ROW_SPEC = pl.BlockSpec((512, 128), lambda i: (i, 0))


def _tc_user_in(x, w, du_p):
    def body(x_ref, w_ref, d_ref, o_ref):
        au = _au_of(d_ref)
        h = jnp.dot(x_ref[...], w_ref[...], preferred_element_type=jnp.float32)
        o_ref[...] = h * au[:, None]

    return pl.pallas_call(
        body, grid=(NUP // 512,),
        in_specs=[_ROW_SPEC, _W_SPEC, _D_SPEC],
        out_specs=_ROW_SPEC,
        out_shape=jax.ShapeDtypeStruct((NUP, 128), jnp.float32),
    )(x, w, du_p)


def _tc_rest_in(x, w, dr_p, dns_p):
    def body(x_ref, w_ref, dr_ref, dns_ref, hr_ref, tn_ref, tr_ref):
        ar = _au_of(dr_ref)
        ans = _an_of(dns_ref)
        h = jnp.dot(x_ref[...], w_ref[...], preferred_element_type=jnp.float32)
        hr_ref[...] = h
        tn_ref[...] = h * ans[:, None]
        tr_ref[...] = h * ar[:, None]

    return pl.pallas_call(
        body, grid=(NRP // 512,),
        in_specs=[_ROW_SPEC, _W_SPEC, _D_SPEC, _D_SPEC],
        out_specs=[_ROW_SPEC] * 3,
        out_shape=[jax.ShapeDtypeStruct((NRP, 128), jnp.float32)] * 3,
    )(x, w, dr_p, dns_p)


def _tc_user_mid(gu, du_p, w_rev):
    def body(g_ref, d_ref, w_ref, o_ref):
        au = _au_of(d_ref)
        h = jnp.dot(g_ref[...], w_ref[...],
                    preferred_element_type=jnp.float32)
        hu = jnp.maximum(h * au[:, None], 0.0)
        o_ref[...] = hu * au[:, None]

    return pl.pallas_call(
        body, grid=(NUP // 512,),
        in_specs=[_ROW_SPEC, _D_SPEC, _W_SPEC],
        out_specs=_ROW_SPEC,
        out_shape=jax.ShapeDtypeStruct((NUP, 128), jnp.float32),
    )(gu, du_p, w_rev)


def _tc_user_out(gu, du_p, w_rev, w_out):
    def body(g_ref, d_ref, w_ref, wo_ref, o_ref):
        au = _au_of(d_ref)
        h = jnp.dot(g_ref[...], w_ref[...],
                    preferred_element_type=jnp.float32)
        hu = jnp.maximum(h * au[:, None], 0.0)
        o_ref[...] = jnp.dot(hu, wo_ref[...],
                             preferred_element_type=jnp.float32)

    return pl.pallas_call(
        body, grid=(NU // 400,),
        in_specs=[pl.BlockSpec((400, 128), lambda i: (i, 0)),
                  pl.BlockSpec((2, 400, 16), lambda i: (0, i, 0)),
                  _W_SPEC, _W_SPEC],
        out_specs=pl.BlockSpec((400, 128), lambda i: (i, 0)),
        out_shape=jax.ShapeDtypeStruct((NU, 128), jnp.float32),
    )(gu, du_p, w_rev, w_out)


def _rest_core(gr_ref, gna_ref, gnb_ref, hp_ref, dr_ref, dns_ref, dnd_ref,
               wr_ref, wn_ref):
    ar = _au_of(dr_ref)
    ans = _an_of(dns_ref)
    andd = _an_of(dnd_ref)
    hp = hp_ref[...]
    m1 = jnp.dot(gr_ref[...] * ar[:, None], wr_ref[...],
                 preferred_element_type=jnp.float32)
    gn = gna_ref[...] + gnb_ref[...]
    near_in = gn * andd[:, None] + hp * (ans * andd)[:, None]
    m2 = jnp.dot(near_in, wn_ref[...], preferred_element_type=jnp.float32)
    return jnp.maximum(m1 + m2, 0.0), ar, ans


_REST_IN_SPECS = [_ROW_SPEC, _ROW_SPEC, _ROW_SPEC, _ROW_SPEC,
                  _D_SPEC, _D_SPEC, _D_SPEC, _W_SPEC, _W_SPEC]


def _tc_rest_mid(grev, gna, gnb, hr_prev, dr_p, dns_p, dnd_p, w_rev, w_near):
    def body(gr_ref, gna_ref, gnb_ref, hp_ref, dr_ref, dns_ref, dnd_ref,
             wr_ref, wn_ref, hr_ref, tn_ref, tr_ref):
        hr, ar, ans = _rest_core(gr_ref, gna_ref, gnb_ref, hp_ref, dr_ref,
                                 dns_ref, dnd_ref, wr_ref, wn_ref)
        hr_ref[...] = hr
        tn_ref[...] = hr * ans[:, None]
        tr_ref[...] = hr * ar[:, None]

    return pl.pallas_call(
        body, grid=(NRP // 512,),
        in_specs=_REST_IN_SPECS,
        out_specs=[_ROW_SPEC] * 3,
        out_shape=[jax.ShapeDtypeStruct((NRP, 128), jnp.float32)] * 3,
    )(grev, gna, gnb, hr_prev, dr_p, dns_p, dnd_p, w_rev, w_near)


def _tc_rest_out(grev, gna, gnb, hr_prev, dr_p, dns_p, dnd_p,
                 w_rev, w_near, w_out):
    def body(gr_ref, gna_ref, gnb_ref, hp_ref, dr_ref, dns_ref, dnd_ref,
             wr_ref, wn_ref, wo_ref, o_ref):
        hr, _, _ = _rest_core(gr_ref, gna_ref, gnb_ref, hp_ref, dr_ref,
                              dns_ref, dnd_ref, wr_ref, wn_ref)
        o_ref[...] = jnp.dot(hr, wo_ref[...],
                             preferred_element_type=jnp.float32)

    rs = pl.BlockSpec((400, 128), lambda i: (i, 0))
    ds = pl.BlockSpec((2, 400, 16), lambda i: (0, i, 0))
    return pl.pallas_call(
        body, grid=(NR // 400,),
        in_specs=[rs, rs, rs, rs, ds, ds, ds, _W_SPEC, _W_SPEC, _W_SPEC],
        out_specs=rs,
        out_shape=jax.ShapeDtypeStruct((NR, 128), jnp.float32),
    )(grev, gna, gnb, hr_prev, dr_p, dns_p, dnd_p, w_rev, w_near, w_out)


def kernel(x_user, x_restaurant, W_in_user, W_in_rest, W1_reviews, W1_rev,
           W1_near, W2_reviews, W2_rev, W2_near, W_out_user, W_out_rest,
           edge_index_reviews, edge_index_rev_reviews, edge_index_near):
    i32 = jnp.int32
    su = edge_index_reviews[0].astype(i32)
    sd = edge_index_reviews[1].astype(i32)
    ns = edge_index_near[0].astype(i32)
    nd = edge_index_near[1].astype(i32)
    su = jnp.concatenate([su, jnp.full((ERP - ER,), NU, i32)]).reshape(-1, 128)
    sd = jnp.concatenate([sd, jnp.full((ERP - ER,), NR, i32)]).reshape(-1, 128)
    ns = jnp.concatenate([ns, jnp.full((ENP - EN,), NR, i32)]).reshape(-1, 128)
    nd = jnp.concatenate([nd, jnp.full((ENP - EN,), NR, i32)]).reshape(-1, 128)

    xu = jnp.pad(x_user, ((0, NUP - NU), (0, 0)))
    xr = jnp.pad(x_restaurant, ((0, NRP - NR), (0, 0)))

    z16 = jnp.zeros((512, 16), jnp.float32)

    du_p, dr_p, dns_p, dnd_p = _sc_degrees(su, sd, ns, nd, z16)

    c4 = lambda t: t.reshape(-1, 32)

    ut1 = _tc_user_in(xu, W_in_user, du_p)
    hr0, nt1, rt1 = _tc_rest_in(xr, W_in_rest, dr_p, dns_p)

    r128 = lambda t: t.reshape(NRP, 128)
    grev1, gna1, gnb1, gu1 = _sc_agg_layer(su, sd, ns, nd,
                                           c4(ut1), c4(nt1), c4(rt1))

    ut2 = _tc_user_mid(gu1, du_p, W1_rev)
    hr1, nt2, rt2 = _tc_rest_mid(
        r128(grev1), r128(gna1), r128(gnb1), hr0,
        dr_p, dns_p, dnd_p, W1_reviews, W1_near)

    grev2, gna2, gnb2, gu2 = _sc_agg_layer(su, sd, ns, nd,
                                           c4(ut2), c4(nt2), c4(rt2))

    out_u = _tc_user_out(gu2, du_p, W2_rev, W_out_user)
    out_r = _tc_rest_out(r128(grev2), r128(gna2), r128(gnb2),
                         hr1, dr_p, dns_p, dnd_p,
                         W2_reviews, W2_near, W_out_rest)

    return (out_u, out_r)
